# baseline (speedup 1.0000x reference)
import functools

import jax
import jax.numpy as jnp
from jax import lax
from jax.experimental import pallas as pl
from jax.experimental.pallas import tpu as pltpu
from jax.experimental.pallas import tpu_sc as plsc

_B, _C, _N = 8, 190, 16384
_NC, _NS, _L = 2, 16, 16
_NW = _NC * _NS
_TPW = _N // _NW
_W = 256
_NCHUNK = _TPW // _W
_NG = _W // _L
_NSTRIPE = 4
_CS = _C // _NSTRIPE

_mesh = plsc.VectorSubcoreMesh(core_axis_name="c", subcore_axis_name="s")


def _merge(a, b):
    a1, a2 = a
    b1, b2 = b
    hi = jnp.maximum(a1, b1)
    lo = jnp.maximum(jnp.minimum(a1, b1), jnp.maximum(a2, b2))
    return hi, lo


@functools.partial(
    pl.kernel,
    mesh=_mesh,
    out_type=jax.ShapeDtypeStruct((_NW, _B, _L), jnp.float32),
    scratch_types=[
        pltpu.VMEM((_C, _W), jnp.float32),
        pltpu.VMEM((_C, _W), jnp.float32),
        pltpu.VMEM((_B, _L), jnp.float32),
        pltpu.SemaphoreType.DMA,
        pltpu.SemaphoreType.DMA,
    ],
)
def _sc_conf(sim_hbm, out_hbm, buf0, buf1, acc_v, sem0, sem1):
    wid = lax.axis_index("s") * _NC + lax.axis_index("c")
    tok0 = wid * _TPW
    bufs = (buf0, buf1)
    sems = (sem0, sem1)

    def chunk_copy(t):
        b, h = divmod(t, _NCHUNK)
        base = tok0 + h * _W
        return pltpu.make_async_copy(
            sim_hbm.at[b, :, pl.ds(base, _W)], bufs[t % 2], sems[t % 2]
        )

    chunk_copy(0).start()
    total = _B * _NCHUNK
    neg = jnp.full((_L,), -jnp.inf, jnp.float32)

    for t in range(total):
        chunk_copy(t).wait()
        if t + 1 < total:
            chunk_copy(t + 1).start()
        buf = bufs[t % 2]
        b, h = divmod(t, _NCHUNK)

        def group_body(g, acc, buf=buf):
            sl = pl.ds(g * _L, _L)

            def chan_body(c, carry, buf=buf, sl=sl):
                new = []
                for s in range(_NSTRIPE):
                    v = buf[c * _NSTRIPE + s, sl]
                    m1, m2 = carry[s]
                    m2 = jnp.maximum(m2, jnp.minimum(m1, v))
                    m1 = jnp.maximum(m1, v)
                    new.append((m1, m2))
                return tuple(new)

            init = tuple((neg, neg) for _ in range(_NSTRIPE))
            stripes = lax.fori_loop(0, _CS, chan_body, init)
            m1, m2 = stripes[0]
            for s in range(1, _NSTRIPE):
                m1, m2 = _merge((m1, m2), stripes[s])
            for c in range(_CS * _NSTRIPE, _C):
                v = buf[c, sl]
                m2 = jnp.maximum(m2, jnp.minimum(m1, v))
                m1 = jnp.maximum(m1, v)
            conf = jnp.exp(1.0 - m1 / (m2 + 1e-8))
            return acc + conf

        init_acc = jnp.zeros((_L,), jnp.float32) if h == 0 else acc_v[b, :]
        acc = lax.fori_loop(0, _NG, group_body, init_acc)
        acc_v[b, :] = acc

    pltpu.sync_copy(acc_v, out_hbm.at[wid])


def kernel(sim_mat):
    out = _sc_conf(sim_mat)
    return out.sum(axis=(0, 2)) / _N


# speedup vs baseline: 1.0957x; 1.0957x over previous
import functools

import jax
import jax.numpy as jnp
from jax import lax
from jax.experimental import pallas as pl
from jax.experimental.pallas import tpu as pltpu
from jax.experimental.pallas import tpu_sc as plsc

_B, _C, _N = 8, 190, 16384
_NC, _NS, _L = 2, 16, 16
_NW = _NC * _NS
_TPW = _N // _NW
_W = 256
_NCHUNK = _TPW // _W
_NG = _W // _L
_NSTRIPE = 4
_CS = _C // _NSTRIPE

_mesh = plsc.VectorSubcoreMesh(core_axis_name="c", subcore_axis_name="s")


def _merge(a, b):
    a1, a2 = a
    b1, b2 = b
    hi = jnp.maximum(a1, b1)
    lo = jnp.maximum(jnp.minimum(a1, b1), jnp.maximum(a2, b2))
    return hi, lo


@functools.partial(
    pl.kernel,
    mesh=_mesh,
    out_type=jax.ShapeDtypeStruct((_NW, _B, _L), jnp.float32),
    scratch_types=[
        pltpu.VMEM((_C, _W), jnp.float32),
        pltpu.VMEM((_C, _W), jnp.float32),
        pltpu.VMEM((_B, _L), jnp.float32),
        pltpu.SemaphoreType.DMA,
        pltpu.SemaphoreType.DMA,
    ],
)
def _sc_conf(sim_hbm, out_hbm, buf0, buf1, acc_v, sem0, sem1):
    wid = lax.axis_index("s") * _NC + lax.axis_index("c")
    tok0 = wid * _TPW
    bufs = (buf0, buf1)
    sems = (sem0, sem1)

    def chunk_copy(t):
        b, h = divmod(t, _NCHUNK)
        base = tok0 + h * _W
        return pltpu.make_async_copy(
            sim_hbm.at[b, :, pl.ds(base, _W)], bufs[t % 2], sems[t % 2]
        )

    chunk_copy(0).start()
    total = _B * _NCHUNK
    neg = jnp.full((_L,), -jnp.inf, jnp.float32)

    for t in range(total):
        chunk_copy(t).wait()
        if t + 1 < total:
            chunk_copy(t + 1).start()
        buf = bufs[t % 2]
        b, h = divmod(t, _NCHUNK)

        def group_body(g, acc, buf=buf):
            sl = pl.ds(g * _L, _L)

            def chan_body(c, carry, buf=buf, sl=sl):
                new = []
                for s in range(_NSTRIPE):
                    v = buf[c * _NSTRIPE + s, sl]
                    m1, m2 = carry[s]
                    m2 = jnp.maximum(m2, jnp.minimum(m1, v))
                    m1 = jnp.maximum(m1, v)
                    new.append((m1, m2))
                return tuple(new)

            return acc + buf[0, sl]

        init_acc = jnp.zeros((_L,), jnp.float32) if h == 0 else acc_v[b, :]
        acc = lax.fori_loop(0, _NG, group_body, init_acc)
        acc_v[b, :] = acc

    pltpu.sync_copy(acc_v, out_hbm.at[wid])


def kernel(sim_mat):
    out = _sc_conf(sim_mat)
    return out.sum(axis=(0, 2)) / _N


# speedup vs baseline: 1.1415x; 1.0418x over previous
import functools

import jax
import jax.numpy as jnp
from jax import lax
from jax.experimental import pallas as pl
from jax.experimental.pallas import tpu as pltpu
from jax.experimental.pallas import tpu_sc as plsc

_B, _C, _N = 8, 190, 16384
_NC, _NS, _L = 2, 16, 16
_NW = _NC * _NS

_N_SC = 8192
_TPW = _N_SC // _NW
_W = _TPW
_NG = _W // _L
_NSTRIPE = 4
_CS = _C // _NSTRIPE

_NB_TC = 2048
_N_TC = _N - _N_SC

_mesh = plsc.VectorSubcoreMesh(core_axis_name="c", subcore_axis_name="s")


def _merge(a, b):
    a1, a2 = a
    b1, b2 = b
    hi = jnp.maximum(a1, b1)
    lo = jnp.maximum(jnp.minimum(a1, b1), jnp.maximum(a2, b2))
    return hi, lo


@functools.partial(
    pl.kernel,
    mesh=_mesh,
    out_type=jax.ShapeDtypeStruct((_NW, _B, _L), jnp.float32),
    scratch_types=[
        pltpu.VMEM((_C, _W), jnp.float32),
        pltpu.VMEM((_C, _W), jnp.float32),
        pltpu.VMEM((_B, _L), jnp.float32),
        pltpu.SemaphoreType.DMA,
        pltpu.SemaphoreType.DMA,
    ],
)
def _sc_conf(sim_hbm, out_hbm, buf0, buf1, acc_v, sem0, sem1):
    wid = lax.axis_index("s") * _NC + lax.axis_index("c")
    tok0 = wid * _W
    bufs = (buf0, buf1)
    sems = (sem0, sem1)

    def chunk_copy(b):
        return pltpu.make_async_copy(
            sim_hbm.at[b, :, pl.ds(tok0, _W)], bufs[b % 2], sems[b % 2]
        )

    chunk_copy(0).start()
    neg = jnp.full((_L,), -jnp.inf, jnp.float32)

    for b in range(_B):
        chunk_copy(b).wait()
        if b + 1 < _B:
            chunk_copy(b + 1).start()
        buf = bufs[b % 2]

        def group_body(g, acc, buf=buf):
            sl = pl.ds(g * _L, _L)

            def chan_body(c, carry, buf=buf, sl=sl):
                new = []
                for s in range(_NSTRIPE):
                    v = buf[c * _NSTRIPE + s, sl]
                    m1, m2 = carry[s]
                    m2 = jnp.maximum(m2, jnp.minimum(m1, v))
                    m1 = jnp.maximum(m1, v)
                    new.append((m1, m2))
                return tuple(new)

            init = tuple((neg, neg) for _ in range(_NSTRIPE))
            stripes = lax.fori_loop(0, _CS, chan_body, init)
            m1, m2 = stripes[0]
            for s in range(1, _NSTRIPE):
                m1, m2 = _merge((m1, m2), stripes[s])
            for c in range(_CS * _NSTRIPE, _C):
                v = buf[c, sl]
                m2 = jnp.maximum(m2, jnp.minimum(m1, v))
                m1 = jnp.maximum(m1, v)
            conf = jnp.exp(1.0 - m1 / (m2 + 1e-8))
            return acc + conf

        acc = lax.fori_loop(0, _NG, group_body, jnp.zeros((_L,), jnp.float32))
        acc_v[b, :] = acc

    pltpu.sync_copy(acc_v, out_hbm.at[wid])


def _tc_body(x_ref, out_ref):
    x = x_ref[0]
    m1 = jnp.max(x, axis=0)
    is_max = x == m1[None, :]
    cnt = jnp.sum(is_max.astype(jnp.float32), axis=0)
    neg = jnp.float32(-jnp.inf)
    m2c = jnp.max(jnp.where(is_max, neg, x), axis=0)
    m2 = jnp.where(cnt > 1.0, m1, m2c)
    conf = jnp.exp(1.0 - m1 / (m2 + 1e-8))
    out_ref[0, 0, :] = conf


def kernel(sim_mat):
    sc_out = _sc_conf(sim_mat)

    nblk = _N_TC // _NB_TC
    blk0 = _N_SC // _NB_TC
    tc_conf = pl.pallas_call(
        _tc_body,
        grid=(_B, nblk),
        in_specs=[pl.BlockSpec((1, _C, _NB_TC), lambda b, n: (b, 0, n + blk0))],
        out_specs=pl.BlockSpec((1, 1, _NB_TC), lambda b, n: (b * nblk + n, 0, 0)),
        out_shape=jax.ShapeDtypeStruct((_B * nblk, 1, _NB_TC), jnp.float32),
    )(sim_mat)

    sc_sum = sc_out.sum(axis=(0, 2))
    tc_sum = tc_conf.reshape(_B, nblk * _NB_TC).sum(axis=-1)
    return (sc_sum + tc_sum) / _N
